# baseline (device time: 104747 ns/iter reference)
import jax
import jax.numpy as jnp
from jax import lax
from jax.experimental import pallas as pl
from jax.experimental.pallas import tpu as pltpu

N_DEV = 8


def kernel(x, Wg, Wu, Wd):
    m, k = x.shape
    n = Wd.shape[1]

    def body(x_ref, wg_ref, wu_ref, wd_ref, out_ref, comm_ref, send_sems, recv_sems):
        my = lax.axis_index("i")
        left = lax.rem(my + (N_DEV - 1), N_DEV)
        right = lax.rem(my + 1, N_DEV)

        barrier_sem = pltpu.get_barrier_semaphore()
        for nbr in (left, right):
            pl.semaphore_signal(
                barrier_sem, inc=1,
                device_id=(nbr,), device_id_type=pl.DeviceIdType.MESH,
            )
        pl.semaphore_wait(barrier_sem, 2)

        gate = jnp.dot(x_ref[...], wg_ref[...], preferred_element_type=jnp.float32)
        up = jnp.dot(x_ref[...], wu_ref[...], preferred_element_type=jnp.float32)
        act = gate * (up * jax.nn.sigmoid(up))
        partial = jnp.dot(act, wd_ref[...], preferred_element_type=jnp.float32)

        comm_ref[0] = partial
        out_ref[...] = partial

        for h in range(N_DEV - 1):
            rdma = pltpu.make_async_remote_copy(
                src_ref=comm_ref.at[h],
                dst_ref=comm_ref.at[h + 1],
                send_sem=send_sems.at[h],
                recv_sem=recv_sems.at[h],
                device_id=(right,),
                device_id_type=pl.DeviceIdType.MESH,
            )
            rdma.start()
            rdma.wait()
            out_ref[...] += comm_ref[h + 1]

    return pl.pallas_call(
        body,
        out_shape=jax.ShapeDtypeStruct((m, n), jnp.float32),
        in_specs=[pl.BlockSpec(memory_space=pltpu.VMEM)] * 4,
        out_specs=pl.BlockSpec(memory_space=pltpu.VMEM),
        scratch_shapes=[
            pltpu.VMEM((N_DEV, m, n), jnp.float32),
            pltpu.SemaphoreType.DMA((N_DEV - 1,)),
            pltpu.SemaphoreType.DMA((N_DEV - 1,)),
        ],
        compiler_params=pltpu.CompilerParams(collective_id=0),
    )(x, Wg, Wu, Wd)


# device time: 30090 ns/iter; 3.4811x vs baseline; 3.4811x over previous
import jax
import jax.numpy as jnp
from jax import lax
from jax.experimental import pallas as pl
from jax.experimental.pallas import tpu as pltpu

N_DEV = 8


def kernel(x, Wg, Wu, Wd):
    m, k = x.shape
    n = Wd.shape[1]
    mrow = m // N_DEV

    def body(x_ref, wg_ref, wu_ref, wd_ref, out_ref, partial_ref, rs_ref,
             rs_send, rs_recv, ag_send, ag_recv):
        my = lax.axis_index("i")

        barrier_sem = pltpu.get_barrier_semaphore()
        for o in range(1, N_DEV):
            pl.semaphore_signal(
                barrier_sem, inc=1,
                device_id=(lax.rem(my + o, N_DEV),),
                device_id_type=pl.DeviceIdType.MESH,
            )
        pl.semaphore_wait(barrier_sem, N_DEV - 1)

        gate = jnp.dot(x_ref[...], wg_ref[...], preferred_element_type=jnp.float32)
        up = jnp.dot(x_ref[...], wu_ref[...], preferred_element_type=jnp.float32)
        act = gate * (up * jax.nn.sigmoid(up))
        partial_ref[...] = jnp.dot(act, wd_ref[...], preferred_element_type=jnp.float32)

        rs_rdmas = []
        for o in range(1, N_DEV):
            j = lax.rem(my + o, N_DEV)
            rdma = pltpu.make_async_remote_copy(
                src_ref=partial_ref.at[pl.ds(j * mrow, mrow)],
                dst_ref=rs_ref.at[o - 1],
                send_sem=rs_send.at[o - 1],
                recv_sem=rs_recv.at[o - 1],
                device_id=(j,),
                device_id_type=pl.DeviceIdType.MESH,
            )
            rdma.start()
            rs_rdmas.append(rdma)

        red = partial_ref[pl.ds(my * mrow, mrow)]
        for o in range(1, N_DEV):
            rs_rdmas[o - 1].wait_recv()
            red = red + rs_ref[o - 1]
        out_ref[pl.ds(my * mrow, mrow)] = red

        ag_rdmas = []
        for o in range(1, N_DEV):
            j = lax.rem(my + o, N_DEV)
            rdma = pltpu.make_async_remote_copy(
                src_ref=out_ref.at[pl.ds(my * mrow, mrow)],
                dst_ref=out_ref.at[pl.ds(my * mrow, mrow)],
                send_sem=ag_send.at[o - 1],
                recv_sem=ag_recv.at[o - 1],
                device_id=(j,),
                device_id_type=pl.DeviceIdType.MESH,
            )
            rdma.start()
            ag_rdmas.append(rdma)

        for o in range(1, N_DEV):
            rs_rdmas[o - 1].wait_send()
            ag_rdmas[o - 1].wait_send()
            ag_rdmas[o - 1].wait_recv()

    return pl.pallas_call(
        body,
        out_shape=jax.ShapeDtypeStruct((m, n), jnp.float32),
        in_specs=[pl.BlockSpec(memory_space=pltpu.VMEM)] * 4,
        out_specs=pl.BlockSpec(memory_space=pltpu.VMEM),
        scratch_shapes=[
            pltpu.VMEM((m, n), jnp.float32),
            pltpu.VMEM((N_DEV - 1, mrow, n), jnp.float32),
            pltpu.SemaphoreType.DMA((N_DEV - 1,)),
            pltpu.SemaphoreType.DMA((N_DEV - 1,)),
            pltpu.SemaphoreType.DMA((N_DEV - 1,)),
            pltpu.SemaphoreType.DMA((N_DEV - 1,)),
        ],
        compiler_params=pltpu.CompilerParams(collective_id=0),
    )(x, Wg, Wu, Wd)


# device time: 22242 ns/iter; 4.7094x vs baseline; 1.3528x over previous
import jax
import jax.numpy as jnp
from jax import lax
from jax.experimental import pallas as pl
from jax.experimental.pallas import tpu as pltpu

N_DEV = 8
N_HALF = 2


def kernel(x, Wg, Wu, Wd):
    m, k = x.shape
    h = Wg.shape[1]
    n = Wd.shape[1]
    mrow = m // N_DEV
    nh = n // N_HALF

    def body(x_ref, wg_hbm, wu_hbm, wd_hbm, out_ref, wg_v, wu_v, wd_v,
             pbuf, rs_ref, ag_ref, w_sems, rs_send, rs_recv,
             ag_send, ag_recv):
        my = lax.axis_index("i")

        barrier_sem = pltpu.get_barrier_semaphore()
        for o in range(1, N_DEV):
            pl.semaphore_signal(
                barrier_sem, inc=1,
                device_id=(lax.rem(my + o, N_DEV),),
                device_id_type=pl.DeviceIdType.MESH,
            )

        wg_dma = pltpu.make_async_copy(wg_hbm, wg_v, w_sems.at[0])
        wu_dma = pltpu.make_async_copy(wu_hbm, wu_v, w_sems.at[1])
        wd_dmas = []
        for hv in range(N_HALF):
            wd_dmas.append(pltpu.make_async_copy(
                wd_hbm.at[:, hv * nh:(hv + 1) * nh], wd_v.at[hv],
                w_sems.at[2 + hv]))
        wg_dma.start()
        wu_dma.start()
        for hv in range(N_HALF):
            wd_dmas[hv].start()

        wg_dma.wait()
        gate = jnp.dot(x_ref[...], wg_v[...], preferred_element_type=jnp.float32)
        wu_dma.wait()
        up = jnp.dot(x_ref[...], wu_v[...], preferred_element_type=jnp.float32)
        act = gate * (up * jax.nn.sigmoid(up))

        rs_rdmas = {}
        for hv in range(N_HALF):
            wd_dmas[hv].wait()
            p = jnp.dot(act, wd_v[hv], preferred_element_type=jnp.float32)
            pbuf[hv] = p.astype(jnp.bfloat16)
            if hv == 0:
                pl.semaphore_wait(barrier_sem, N_DEV - 1)
            for o in range(1, N_DEV):
                j = lax.rem(my + o, N_DEV)
                rdma = pltpu.make_async_remote_copy(
                    src_ref=pbuf.at[hv].at[pl.ds(j * mrow, mrow)],
                    dst_ref=rs_ref.at[hv].at[o - 1],
                    send_sem=rs_send.at[hv].at[o - 1],
                    recv_sem=rs_recv.at[hv].at[o - 1],
                    device_id=(j,),
                    device_id_type=pl.DeviceIdType.MESH,
                )
                rdma.start()
                rs_rdmas[hv, o] = rdma

        ag_rdmas = {}
        for hv in range(N_HALF):
            red = pbuf[hv, pl.ds(my * mrow, mrow)].astype(jnp.float32)
            for o in range(1, N_DEV):
                rs_rdmas[hv, o].wait_recv()
                red = red + rs_ref[hv, o - 1].astype(jnp.float32)
            out_ref[pl.ds(my * mrow, mrow), hv * nh:(hv + 1) * nh] = red
            ag_ref[hv, my] = red.astype(jnp.bfloat16)
            for o in range(1, N_DEV):
                j = lax.rem(my + o, N_DEV)
                rdma = pltpu.make_async_remote_copy(
                    src_ref=ag_ref.at[hv].at[my],
                    dst_ref=ag_ref.at[hv].at[my],
                    send_sem=ag_send.at[hv].at[o - 1],
                    recv_sem=ag_recv.at[hv].at[o - 1],
                    device_id=(j,),
                    device_id_type=pl.DeviceIdType.MESH,
                )
                rdma.start()
                ag_rdmas[hv, o] = rdma

        for hv in range(N_HALF):
            for o in range(1, N_DEV):
                j = lax.rem(my + (N_DEV - o), N_DEV)
                ag_rdmas[hv, o].wait_recv()
                out_ref[pl.ds(j * mrow, mrow), hv * nh:(hv + 1) * nh] = (
                    ag_ref[hv, j].astype(jnp.float32))

        for hv in range(N_HALF):
            for o in range(1, N_DEV):
                rs_rdmas[hv, o].wait_send()
                ag_rdmas[hv, o].wait_send()

    return pl.pallas_call(
        body,
        out_shape=jax.ShapeDtypeStruct((m, n), jnp.float32),
        in_specs=[
            pl.BlockSpec(memory_space=pltpu.VMEM),
            pl.BlockSpec(memory_space=pltpu.MemorySpace.HBM),
            pl.BlockSpec(memory_space=pltpu.MemorySpace.HBM),
            pl.BlockSpec(memory_space=pltpu.MemorySpace.HBM),
        ],
        out_specs=pl.BlockSpec(memory_space=pltpu.VMEM),
        scratch_shapes=[
            pltpu.VMEM((k, h), jnp.float32),
            pltpu.VMEM((k, h), jnp.float32),
            pltpu.VMEM((N_HALF, h, nh), jnp.float32),
            pltpu.VMEM((N_HALF, m, nh), jnp.bfloat16),
            pltpu.VMEM((N_HALF, N_DEV - 1, mrow, nh), jnp.bfloat16),
            pltpu.VMEM((N_HALF, N_DEV, mrow, nh), jnp.bfloat16),
            pltpu.SemaphoreType.DMA((2 + N_HALF,)),
            pltpu.SemaphoreType.DMA((N_HALF, N_DEV - 1)),
            pltpu.SemaphoreType.DMA((N_HALF, N_DEV - 1)),
            pltpu.SemaphoreType.DMA((N_HALF, N_DEV - 1)),
            pltpu.SemaphoreType.DMA((N_HALF, N_DEV - 1)),
        ],
        compiler_params=pltpu.CompilerParams(collective_id=0),
    )(x, Wg, Wu, Wd)
